# baseline (speedup 1.0000x reference)
import functools

import jax
import jax.numpy as jnp
from jax import lax
from jax.experimental import pallas as pl
from jax.experimental.pallas import tpu as pltpu
from jax.experimental.pallas import tpu_sc as plsc

B = 16384
D = 16
VOCAB = 1000000
RPB = 128 // D
NBLK = VOCAB // RPB
NC = 2
NS = 16
NW = NC * NS
BPW = B // NW
CH = 128
NCH = BPW // CH
HB = 256
EPS2 = 1e-16


def _rsqrt(x):
    i = lax.bitcast_convert_type(x, jnp.int32)
    i = jnp.int32(0x5F3759DF) - lax.shift_right_logical(i, 1)
    y = lax.bitcast_convert_type(i, jnp.float32)
    for _ in range(3):
        y = y * (1.5 - 0.5 * x * y * y)
    return y


def _body(uidx_hbm, midx_hbm, utab_hbm, mtab_hbm, out_hbm,
          uidx_v, midx_v, ublkidx_v, mblkidx_v, ublk_v, mblk_v, out_v, sem):
    wid = lax.axis_index("s") * NC + lax.axis_index("c")
    base = wid * BPW

    for j in range(NCH):
        pltpu.sync_copy(uidx_hbm.at[pl.ds(base + j * CH, CH)], uidx_v.at[j])
        pltpu.sync_copy(midx_hbm.at[pl.ds(base + j * CH, CH)], midx_v.at[j])

    for j in range(NCH):
        for o in range(CH // 16):
            s = pl.ds(o * 16, 16)
            ui = uidx_v[j, s]
            mi = midx_v[j, s]
            ublkidx_v[j, s] = lax.shift_right_logical(ui, 3)
            mblkidx_v[j, s] = lax.shift_right_logical(mi, 3)
            uidx_v[j, s] = (ui & 7) * D
            midx_v[j, s] = (mi & 7) * D

    lanes = lax.iota(jnp.int32, 16)

    for h in range(BPW // HB):
        copies = []
        for j in range(HB // CH):
            jj = h * (HB // CH) + j
            copies.append(pltpu.async_copy(
                utab_hbm.at[ublkidx_v.at[jj]], ublk_v.at[pl.ds(j * CH, CH)],
                sem))
            copies.append(pltpu.async_copy(
                mtab_hbm.at[mblkidx_v.at[jj]], mblk_v.at[pl.ds(j * CH, CH)],
                sem))
        for c in copies:
            c.wait()

        def group(g, carry, h=h):
            ridx = g * 16 + lanes
            jj = h * (HB // CH) + g // 8
            s = pl.ds((g % 8) * 16, 16)
            usub = uidx_v[jj, s]
            msub = midx_v[jj, s]
            acc_um = jnp.zeros((16,), jnp.float32)
            acc_uu = jnp.zeros((16,), jnp.float32)
            acc_mm = jnp.zeros((16,), jnp.float32)
            for d in range(D):
                u = plsc.load_gather(ublk_v, [ridx, usub + d])
                m = plsc.load_gather(mblk_v, [ridx, msub + d])
                acc_um = acc_um + u * m
                acc_uu = acc_uu + u * u
                acc_mm = acc_mm + m * m
            denom2 = jnp.maximum(acc_uu, EPS2) * jnp.maximum(acc_mm, EPS2)
            sim = acc_um * _rsqrt(denom2) * 2.5 + 2.75
            out_v[pl.ds(h * HB + g * 16, 16)] = sim
            return carry

        lax.fori_loop(0, HB // 16, group, jnp.int32(0))

    pltpu.sync_copy(out_v, out_hbm.at[pl.ds(base, BPW)])


_mesh = plsc.VectorSubcoreMesh(core_axis_name="c", subcore_axis_name="s")

_sc_call = functools.partial(
    pl.kernel,
    mesh=_mesh,
    compiler_params=pltpu.CompilerParams(needs_layout_passes=False),
    out_type=jax.ShapeDtypeStruct((B,), jnp.float32),
    scratch_types=[
        pltpu.VMEM((NCH, CH), jnp.int32),
        pltpu.VMEM((NCH, CH), jnp.int32),
        pltpu.VMEM((NCH, CH), jnp.int32),
        pltpu.VMEM((NCH, CH), jnp.int32),
        pltpu.VMEM((HB, 128), jnp.float32),
        pltpu.VMEM((HB, 128), jnp.float32),
        pltpu.VMEM((BPW,), jnp.float32),
        pltpu.SemaphoreType.DMA,
    ],
)(_body)


def kernel(user_idx, movie_idx, user_table, movie_table):
    return _sc_call(user_idx.astype(jnp.int32), movie_idx.astype(jnp.int32),
                    user_table.reshape(NBLK, 128),
                    movie_table.reshape(NBLK, 128))


# speedup vs baseline: 1.0105x; 1.0105x over previous
import functools

import jax
import jax.numpy as jnp
from jax import lax
from jax.experimental import pallas as pl
from jax.experimental.pallas import tpu as pltpu
from jax.experimental.pallas import tpu_sc as plsc

B = 16384
D = 16
VOCAB = 1000000
NBLK = VOCAB * D // 128
NC = 2
NS = 16
NW = NC * NS
BPW = B // NW
CH = 128
NCH = BPW // CH
EPS2 = 1e-16


def _rsqrt(x):
    i = lax.bitcast_convert_type(x, jnp.int32)
    i = jnp.int32(0x5F3759DF) - lax.shift_right_logical(i, 1)
    y = lax.bitcast_convert_type(i, jnp.float32)
    for _ in range(3):
        y = y * (1.5 - 0.5 * x * y * y)
    return y


def _body(uidx_hbm, midx_hbm, utab_hbm, mtab_hbm, out_hbm,
          uidx_v, midx_v, ublkidx_v, mblkidx_v, ublk_v, mblk_v, out_v,
          sem, isem):
    wid = lax.axis_index("s") * NC + lax.axis_index("c")
    base = wid * BPW

    icopies = []
    for j in range(NCH):
        icopies.append(pltpu.async_copy(
            uidx_hbm.at[pl.ds(base + j * CH, CH)], uidx_v.at[j], isem))
        icopies.append(pltpu.async_copy(
            midx_hbm.at[pl.ds(base + j * CH, CH)], midx_v.at[j], isem))
    for c in icopies:
        c.wait()

    for j in range(NCH):
        for o in range(CH // 16):
            s = pl.ds(o * 16, 16)
            ui = uidx_v[j, s]
            mi = midx_v[j, s]
            ublkidx_v[j, s] = lax.shift_right_logical(ui, 3)
            mblkidx_v[j, s] = lax.shift_right_logical(mi, 3)
            uidx_v[j, s] = (ui & 7) * D
            midx_v[j, s] = (mi & 7) * D

    lanes = lax.iota(jnp.int32, 16)

    def fire(q):
        buf = q % 2
        return (pltpu.async_copy(
                    utab_hbm.at[ublkidx_v.at[q]],
                    ublk_v.at[pl.ds(buf * CH, CH)], sem),
                pltpu.async_copy(
                    mtab_hbm.at[mblkidx_v.at[q]],
                    mblk_v.at[pl.ds(buf * CH, CH)], sem))

    inflight = fire(0)
    for q in range(NCH):
        for c in inflight:
            c.wait()
        if q + 1 < NCH:
            inflight = fire(q + 1)
        roff = (q % 2) * CH

        def group(g, carry, q=q, roff=roff):
            ridx = roff + g * 16 + lanes
            s = pl.ds(g * 16, 16)
            usub = uidx_v[q, s]
            msub = midx_v[q, s]
            acc_um = jnp.zeros((16,), jnp.float32)
            acc_uu = jnp.zeros((16,), jnp.float32)
            acc_mm = jnp.zeros((16,), jnp.float32)
            for d in range(D):
                u = plsc.load_gather(ublk_v, [ridx, usub + d])
                m = plsc.load_gather(mblk_v, [ridx, msub + d])
                acc_um = acc_um + u * m
                acc_uu = acc_uu + u * u
                acc_mm = acc_mm + m * m
            denom2 = jnp.maximum(acc_uu, EPS2) * jnp.maximum(acc_mm, EPS2)
            sim = acc_um * _rsqrt(denom2) * 2.5 + 2.75
            out_v[pl.ds(q * CH + g * 16, 16)] = sim
            return carry

        lax.fori_loop(0, CH // 16, group, jnp.int32(0))

    pltpu.sync_copy(out_v, out_hbm.at[pl.ds(base, BPW)])


_mesh = plsc.VectorSubcoreMesh(core_axis_name="c", subcore_axis_name="s")

_sc_call = functools.partial(
    pl.kernel,
    mesh=_mesh,
    compiler_params=pltpu.CompilerParams(needs_layout_passes=False),
    out_type=jax.ShapeDtypeStruct((B,), jnp.float32),
    scratch_types=[
        pltpu.VMEM((NCH, CH), jnp.int32),
        pltpu.VMEM((NCH, CH), jnp.int32),
        pltpu.VMEM((NCH, CH), jnp.int32),
        pltpu.VMEM((NCH, CH), jnp.int32),
        pltpu.VMEM((2 * CH, 128), jnp.float32),
        pltpu.VMEM((2 * CH, 128), jnp.float32),
        pltpu.VMEM((BPW,), jnp.float32),
        pltpu.SemaphoreType.DMA,
        pltpu.SemaphoreType.DMA,
    ],
)(_body)


def kernel(user_idx, movie_idx, user_table, movie_table):
    return _sc_call(user_idx.astype(jnp.int32), movie_idx.astype(jnp.int32),
                    user_table.reshape(NBLK, 128),
                    movie_table.reshape(NBLK, 128))


# speedup vs baseline: 1.4315x; 1.4167x over previous
import functools

import jax
import jax.numpy as jnp
from jax import lax
from jax.experimental import pallas as pl
from jax.experimental.pallas import tpu as pltpu
from jax.experimental.pallas import tpu_sc as plsc

B = 16384
D = 16
NC = 2
NS = 16
NW = NC * NS
BPW = B // NW
G = 16
NG = BPW // G
EPS2 = 1e-16


def _rsqrt(x):
    i = lax.bitcast_convert_type(x, jnp.int32)
    i = jnp.int32(0x5F3759DF) - lax.shift_right_logical(i, 1)
    y = lax.bitcast_convert_type(i, jnp.float32)
    for _ in range(3):
        y = y * (1.5 - 0.5 * x * y * y)
    return y


def _body(uidx_hbm, midx_hbm, utab_hbm, mtab_hbm, out_hbm,
          uidx_v, midx_v, ublk_v, mblk_v, uring_v, mring_v, out_v,
          sem, isem):
    wid = lax.axis_index("s") * NC + lax.axis_index("c")
    base = wid * BPW

    icopies = []
    for j in range(BPW // 128):
        icopies.append(pltpu.async_copy(
            uidx_hbm.at[pl.ds(base + j * 128, 128)],
            uidx_v.at[pl.ds(j * 128, 128)], isem))
        icopies.append(pltpu.async_copy(
            midx_hbm.at[pl.ds(base + j * 128, 128)],
            midx_v.at[pl.ds(j * 128, 128)], isem))
    for c in icopies:
        c.wait()

    for o in range(BPW // 16):
        s = pl.ds(o * 16, 16)
        ui = uidx_v[s]
        mi = midx_v[s]
        ublk_v[s] = ui & ~jnp.int32(7)
        mblk_v[s] = mi & ~jnp.int32(7)
        uidx_v[s] = ui & 7
        midx_v[s] = mi & 7

    lanes = lax.iota(jnp.int32, 16)

    def fire(g, bank):
        ubv = ublk_v[pl.ds(g * G, 16)]
        mbv = mblk_v[pl.ds(g * G, 16)]

        def one(j, carry):
            msk = lanes == j
            ub = lax.reduce_max(jnp.where(msk, ubv, 0), axes=(0,))
            mb = lax.reduce_max(jnp.where(msk, mbv, 0), axes=(0,))
            ub = pl.multiple_of(ub, 8)
            mb = pl.multiple_of(mb, 8)
            pltpu.async_copy(
                utab_hbm.at[pl.ds(ub, 8)], uring_v.at[bank * G + j], sem)
            pltpu.async_copy(
                mtab_hbm.at[pl.ds(mb, 8)], mring_v.at[bank * G + j], sem)
            return carry

        lax.fori_loop(0, G, one, jnp.int32(0))

    def compute(g, bank):
        s = pl.ds(g * G, 16)
        usub = uidx_v[s]
        msub = midx_v[s]
        bufv = bank * G + lanes
        acc_um = jnp.zeros((16,), jnp.float32)
        acc_uu = jnp.zeros((16,), jnp.float32)
        acc_mm = jnp.zeros((16,), jnp.float32)
        for d in range(D):
            dv = jnp.full((16,), d, jnp.int32)
            u = plsc.load_gather(uring_v, [bufv, usub, dv])
            m = plsc.load_gather(mring_v, [bufv, msub, dv])
            acc_um = acc_um + u * m
            acc_uu = acc_uu + u * u
            acc_mm = acc_mm + m * m
        denom2 = jnp.maximum(acc_uu, EPS2) * jnp.maximum(acc_mm, EPS2)
        sim = acc_um * _rsqrt(denom2) * 2.5 + 2.75
        out_v[pl.ds(g * G, 16)] = sim

    fire(jnp.int32(0), jnp.int32(0))
    fire(jnp.int32(1), jnp.int32(1))

    def step(g, carry):
        bank = lax.rem(g, 2)
        for j in range(G):
            pltpu.make_async_copy(
                utab_hbm.at[pl.ds(0, 8)],
                uring_v.at[bank * G + j], sem).wait()
            pltpu.make_async_copy(
                mtab_hbm.at[pl.ds(0, 8)],
                mring_v.at[bank * G + j], sem).wait()
        compute(g, bank)

        @pl.when(g + 2 < NG)
        def _():
            fire(g + 2, bank)

        return carry

    lax.fori_loop(0, NG, step, jnp.int32(0))

    pltpu.sync_copy(out_v, out_hbm.at[pl.ds(base, BPW)])


_mesh = plsc.VectorSubcoreMesh(core_axis_name="c", subcore_axis_name="s")

_sc_call = functools.partial(
    pl.kernel,
    mesh=_mesh,
    compiler_params=pltpu.CompilerParams(needs_layout_passes=False),
    out_type=jax.ShapeDtypeStruct((B,), jnp.float32),
    scratch_types=[
        pltpu.VMEM((BPW,), jnp.int32),
        pltpu.VMEM((BPW,), jnp.int32),
        pltpu.VMEM((BPW,), jnp.int32),
        pltpu.VMEM((BPW,), jnp.int32),
        pltpu.VMEM((2 * G, 8, D), jnp.float32),
        pltpu.VMEM((2 * G, 8, D), jnp.float32),
        pltpu.VMEM((BPW,), jnp.float32),
        pltpu.SemaphoreType.DMA,
        pltpu.SemaphoreType.DMA,
    ],
)(_body)


def kernel(user_idx, movie_idx, user_table, movie_table):
    return _sc_call(user_idx.astype(jnp.int32), movie_idx.astype(jnp.int32),
                    user_table, movie_table)
